# baseline (speedup 1.0000x reference)
import functools

import jax
import jax.numpy as jnp
from jax import lax
from jax.experimental import pallas as pl
from jax.experimental.pallas import tpu as pltpu
from jax.experimental.pallas import tpu_sc as plsc

_TN = 512
_TN2 = 16
_TAIL = 1024


def _head_block(nodes_ref, mb0_ref, mb1_ref, wl_ref, wr_ref, wt_ref, b_ref,
                out_ref, *, c):
    half = c // 2
    x0 = mb0_ref[...]
    x1 = mb1_ref[...]
    inv = 1.0 / (c - 1)
    cf = jax.lax.broadcasted_iota(jnp.int32, (1, half, 1), 1).astype(jnp.float32)
    s = jnp.sum(x0 + x1, axis=1)
    a = jnp.sum((cf * inv) * (x0 + x1) + (half * inv) * x1, axis=1)
    wl = wl_ref[...]
    acc = jnp.dot(s, wl, preferred_element_type=jnp.float32)
    acc += jnp.dot(a, wr_ref[...] - wl, preferred_element_type=jnp.float32)
    acc += jnp.dot(nodes_ref[...], wt_ref[...], preferred_element_type=jnp.float32)
    out_ref[...] = jnp.maximum(acc + b_ref[...], 0.0)


def _tail_block(y_ref, s_ref, a_ref, nodes_ref, wl_ref, wr_ref, wt_ref, b_ref,
                out_ref):
    del y_ref
    wl = wl_ref[...]
    acc = jnp.dot(s_ref[...], wl, preferred_element_type=jnp.float32)
    acc += jnp.dot(a_ref[...], wr_ref[...] - wl, preferred_element_type=jnp.float32)
    acc += jnp.dot(nodes_ref[...], wt_ref[...], preferred_element_type=jnp.float32)
    out_ref[...] = jnp.maximum(acc + b_ref[...], 0.0)


def _make_sc_reduce(c, h, off, tail):
    info = plsc.get_sparse_core_info()
    nw = info.num_cores * info.num_subcores
    rpw = tail // nw
    inv = 1.0 / (c - 1)
    nhc = h // 16

    mesh = plsc.VectorSubcoreMesh(core_axis_name="c", subcore_axis_name="s")

    @functools.partial(
        pl.kernel,
        out_type=(
            jax.ShapeDtypeStruct((tail, h), jnp.float32),
            jax.ShapeDtypeStruct((tail, h), jnp.float32),
        ),
        mesh=mesh,
        scratch_types=[
            pltpu.VMEM((2, c, h), jnp.float32),
            pltpu.VMEM((rpw, h), jnp.float32),
            pltpu.VMEM((rpw, h), jnp.float32),
            pltpu.SemaphoreType.DMA,
            pltpu.SemaphoreType.DMA,
        ],
    )
    def sc_reduce(mb_hbm, s_hbm, a_hbm, mb_v, s_v, a_v, sem0, sem1):
        wid = lax.axis_index("s") * info.num_cores + lax.axis_index("c")
        base = off + wid * rpw

        def reduce_row(buf, row):
            for hi in range(nhc):
                sl = pl.ds(hi * 16, 16)
                x = mb_v[buf, 0, sl]
                s_acc = x
                a_acc = (1.0 * inv) * mb_v[buf, 1, sl]
                s_acc = s_acc + mb_v[buf, 1, sl]
                for ci in range(2, c):
                    x = mb_v[buf, ci, sl]
                    s_acc = s_acc + x
                    a_acc = a_acc + (ci * inv) * x
                s_v[row, sl] = s_acc
                a_v[row, sl] = a_acc

        pltpu.make_async_copy(mb_hbm.at[base], mb_v.at[0], sem0).start()

        def body(i, carry):
            r0 = 2 * i
            pltpu.make_async_copy(mb_hbm.at[base + r0 + 1], mb_v.at[1], sem1).start()
            pltpu.make_async_copy(mb_hbm.at[base + r0], mb_v.at[0], sem0).wait()
            reduce_row(0, r0)

            @pl.when(r0 + 2 < rpw)
            def _():
                pltpu.make_async_copy(
                    mb_hbm.at[base + r0 + 2], mb_v.at[0], sem0).start()

            pltpu.make_async_copy(mb_hbm.at[base + r0 + 1], mb_v.at[1], sem1).wait()
            reduce_row(1, r0 + 1)
            return carry

        lax.fori_loop(0, rpw // 2, body, 0)

        out_sl = pl.ds(wid * rpw, rpw)
        pltpu.sync_copy(s_v, s_hbm.at[out_sl])
        pltpu.sync_copy(a_v, a_hbm.at[out_sl])

    return sc_reduce


def kernel(nodes_h, mailbox_h, W_left, W_right, W_top, b_conv):
    n, c, h = mailbox_h.shape
    head = n - _TAIL
    tb = head // _TN2

    s_tail, a_tail = _make_sc_reduce(c, h, head, _TAIL)(mailbox_h)

    y = pl.pallas_call(
        functools.partial(_head_block, c=c),
        grid=(pl.cdiv(head, _TN),),
        in_specs=[
            pl.BlockSpec((_TN, h), lambda i: (i, 0)),
            pl.BlockSpec((_TN, c // 2, h), lambda i: (i, 0, 0)),
            pl.BlockSpec((_TN, c // 2, h), lambda i: (i, 1, 0)),
            pl.BlockSpec((h, h), lambda i: (0, 0)),
            pl.BlockSpec((h, h), lambda i: (0, 0)),
            pl.BlockSpec((h, h), lambda i: (0, 0)),
            pl.BlockSpec((1, h), lambda i: (0, 0)),
        ],
        out_specs=pl.BlockSpec((_TN, h), lambda i: (i, 0)),
        out_shape=jax.ShapeDtypeStruct((n, h), jnp.float32),
        compiler_params=pltpu.CompilerParams(
            dimension_semantics=("parallel",),
        ),
    )(nodes_h, mailbox_h, mailbox_h, W_left, W_right, W_top, b_conv)

    return pl.pallas_call(
        _tail_block,
        grid=(_TAIL // _TN2,),
        in_specs=[
            pl.BlockSpec(memory_space=pl.ANY),
            pl.BlockSpec((_TN2, h), lambda i: (i, 0)),
            pl.BlockSpec((_TN2, h), lambda i: (i, 0)),
            pl.BlockSpec((_TN2, h), lambda i: (tb + i, 0)),
            pl.BlockSpec((h, h), lambda i: (0, 0)),
            pl.BlockSpec((h, h), lambda i: (0, 0)),
            pl.BlockSpec((h, h), lambda i: (0, 0)),
            pl.BlockSpec((1, h), lambda i: (0, 0)),
        ],
        out_specs=pl.BlockSpec((_TN2, h), lambda i: (tb + i, 0)),
        out_shape=jax.ShapeDtypeStruct((n, h), jnp.float32),
        input_output_aliases={0: 0},
        compiler_params=pltpu.CompilerParams(
            dimension_semantics=("arbitrary",),
        ),
    )(y, s_tail, a_tail, nodes_h, W_left, W_right, W_top, b_conv)


# speedup vs baseline: 1.7329x; 1.7329x over previous
import functools

import jax
import jax.numpy as jnp
from jax.experimental import pallas as pl
from jax.experimental.pallas import tpu as pltpu

_TN = 512


def _tbcnn_block(nodes_ref, mb_ref, wl_ref, wr_ref, wt_ref, b_ref, out_ref,
                 *, c):
    mb = mb_ref[...]
    coef = (jax.lax.broadcasted_iota(jnp.int32, (1, c, 1), 1)
            .astype(jnp.float32)) / (c - 1)
    s = jnp.sum(mb, axis=1)
    a = jnp.sum(mb * coef, axis=1)
    wl = wl_ref[...]
    acc = jnp.dot(s, wl, preferred_element_type=jnp.float32)
    acc += jnp.dot(a, wr_ref[...] - wl, preferred_element_type=jnp.float32)
    acc += jnp.dot(nodes_ref[...], wt_ref[...], preferred_element_type=jnp.float32)
    out_ref[...] = jnp.maximum(acc + b_ref[...], 0.0)


def kernel(nodes_h, mailbox_h, W_left, W_right, W_top, b_conv):
    n, c, h = mailbox_h.shape
    return pl.pallas_call(
        functools.partial(_tbcnn_block, c=c),
        grid=(pl.cdiv(n, _TN),),
        in_specs=[
            pl.BlockSpec((_TN, h), lambda i: (i, 0)),
            pl.BlockSpec((_TN, c, h), lambda i: (i, 0, 0)),
            pl.BlockSpec((h, h), lambda i: (0, 0)),
            pl.BlockSpec((h, h), lambda i: (0, 0)),
            pl.BlockSpec((h, h), lambda i: (0, 0)),
            pl.BlockSpec((1, h), lambda i: (0, 0)),
        ],
        out_specs=pl.BlockSpec((_TN, h), lambda i: (i, 0)),
        out_shape=jax.ShapeDtypeStruct((n, h), jnp.float32),
        compiler_params=pltpu.CompilerParams(
            dimension_semantics=("parallel",),
        ),
    )(nodes_h, mailbox_h, W_left, W_right, W_top, b_conv)


# speedup vs baseline: 1.8523x; 1.0689x over previous
import functools

import jax
import jax.numpy as jnp
from jax.experimental import pallas as pl
from jax.experimental.pallas import tpu as pltpu

_TN = 1024


def _tbcnn_block(nodes_ref, mb_ref, wl_ref, wr_ref, wt_ref, b_ref, out_ref,
                 *, c):
    mb = mb_ref[...]
    coef = (jax.lax.broadcasted_iota(jnp.int32, (1, c, 1), 1)
            .astype(jnp.float32)) / (c - 1)
    s = jnp.sum(mb, axis=1)
    a = jnp.sum(mb * coef, axis=1)
    wl = wl_ref[...]
    acc = jnp.dot(s, wl, preferred_element_type=jnp.float32)
    acc += jnp.dot(a, wr_ref[...] - wl, preferred_element_type=jnp.float32)
    acc += jnp.dot(nodes_ref[...], wt_ref[...], preferred_element_type=jnp.float32)
    out_ref[...] = jnp.maximum(acc + b_ref[...], 0.0)


def kernel(nodes_h, mailbox_h, W_left, W_right, W_top, b_conv):
    n, c, h = mailbox_h.shape
    return pl.pallas_call(
        functools.partial(_tbcnn_block, c=c),
        grid=(pl.cdiv(n, _TN),),
        in_specs=[
            pl.BlockSpec((_TN, h), lambda i: (i, 0)),
            pl.BlockSpec((_TN, c, h), lambda i: (i, 0, 0)),
            pl.BlockSpec((h, h), lambda i: (0, 0)),
            pl.BlockSpec((h, h), lambda i: (0, 0)),
            pl.BlockSpec((h, h), lambda i: (0, 0)),
            pl.BlockSpec((1, h), lambda i: (0, 0)),
        ],
        out_specs=pl.BlockSpec((_TN, h), lambda i: (i, 0)),
        out_shape=jax.ShapeDtypeStruct((n, h), jnp.float32),
        compiler_params=pltpu.CompilerParams(
            dimension_semantics=("parallel",),
        ),
    )(nodes_h, mailbox_h, W_left, W_right, W_top, b_conv)
